# Initial kernel scaffold; baseline (speedup 1.0000x reference)
import jax
import jax.numpy as jnp
from jax.experimental import pallas as pl


def kernel(position_ids):
    raise NotImplementedError("write your pallas kernel here")


# speedup vs baseline: 1377.8942x; 1377.8942x over previous
import functools

import jax
import jax.numpy as jnp
from jax import lax
from jax.experimental import pallas as pl

HEAD_DIM = 128
BASE = 10000
FREQ_ALLOCATION = 20
HALF = HEAD_DIM // 2
SPLIT = HALF - FREQ_ALLOCATION

S_BLK = 512


def _rope_body(pid_ref, freq_ref, out_ref):
    pid = pid_ref[0]
    pt = pid[:, 0:1].astype(jnp.float32)
    ph = pid[:, 1:2].astype(jnp.float32)
    pw = pid[:, 2:3].astype(jnp.float32)
    c = lax.broadcasted_iota(jnp.int32, (S_BLK, HEAD_DIM), 1)
    j = c // 2
    use_hw = j < SPLIT
    use_h = (j % 2) == 0
    pos = jnp.where(use_hw, jnp.where(use_h, ph, pw), pt)
    ang = pos * freq_ref[...]
    out_ref[0, 0] = jnp.sin(ang)
    out_ref[1, 0] = jnp.cos(ang)


def kernel(position_ids):
    B, S, _ = position_ids.shape
    idx = jnp.arange(0, HEAD_DIM, 2, dtype=jnp.float32)
    inv_freq = 1.0 / (BASE ** (idx / HEAD_DIM))
    freq128 = jnp.repeat(inv_freq, 2).reshape(1, HEAD_DIM)

    out = pl.pallas_call(
        _rope_body,
        grid=(B, S // S_BLK),
        in_specs=[
            pl.BlockSpec((1, S_BLK, 3), lambda b, i: (b, i, 0)),
            pl.BlockSpec((1, HEAD_DIM), lambda b, i: (0, 0)),
        ],
        out_specs=pl.BlockSpec((2, 1, S_BLK, HEAD_DIM), lambda b, i: (0, b, i, 0)),
        out_shape=jax.ShapeDtypeStruct((2, B, S, HEAD_DIM), jnp.float32),
    )(position_ids, freq128)
    return out.reshape(2 * B, S, 1, HEAD_DIM)
